# baseline (speedup 1.0000x reference)
import math

import jax
import jax.numpy as jnp
from jax import lax
from jax.experimental import pallas as pl
from jax.experimental.pallas import tpu as pltpu
from jax.experimental.pallas import tpu_sc as plsc

_D = 64
_SCALE = math.sqrt(_D)
_NC = 2
_NS = 16
_NW = _NC * _NS
_CHUNK = 128


def _emb_body(table_hbm, idx_hbm, out_hbm, idx_v, rows_v, gsem):
    wid = lax.axis_index("s") * _NC + lax.axis_index("c")
    nchunk = idx_hbm.shape[1]
    pltpu.sync_copy(idx_hbm.at[wid], idx_v)

    def chunk_body(c, carry):
        pltpu.async_copy(table_hbm.at[idx_v.at[c]], rows_v, gsem).wait()

        def scale_body(r, carry2):
            for j in range(_D // 16):
                s = pl.ds(j * 16, 16)
                rows_v[r, s] = rows_v[r, s] * _SCALE
            return carry2

        lax.fori_loop(0, _CHUNK, scale_body, 0, unroll=4)
        base = (wid * nchunk + c) * _CHUNK
        pltpu.sync_copy(rows_v, out_hbm.at[pl.ds(base, _CHUNK)])
        return carry

    lax.fori_loop(0, nchunk, chunk_body, 0)


def kernel(x, emb_weight):
    bt, seq = x.shape
    b = bt * seq
    assert b % (_NW * _CHUNK) == 0
    nchunk = b // (_NW * _CHUNK)
    idx = x.reshape(_NW, nchunk, _CHUNK).astype(jnp.int32)

    mesh = plsc.VectorSubcoreMesh(core_axis_name="c", subcore_axis_name="s")
    f = pl.kernel(
        _emb_body,
        mesh=mesh,
        out_type=jax.ShapeDtypeStruct((b, _D), jnp.float32),
        scratch_types=[
            pltpu.VMEM((nchunk, _CHUNK), jnp.int32),
            pltpu.VMEM((_CHUNK, _D), jnp.float32),
            pltpu.SemaphoreType.DMA,
        ],
        compiler_params=pltpu.CompilerParams(use_tc_tiling_on_sc=False),
    )
    out = f(emb_weight, idx)
    return out.reshape(bt, seq, _D)


# speedup vs baseline: 1.0870x; 1.0870x over previous
import math

import jax
import jax.numpy as jnp
from jax import lax
from jax.experimental import pallas as pl
from jax.experimental.pallas import tpu as pltpu
from jax.experimental.pallas import tpu_sc as plsc

_D = 64
_SCALE = math.sqrt(_D)
_NC = 2
_NS = 16
_NW = _NC * _NS
_CHUNK = 128
_NBUF = 4


def _emb_body(table_hbm, idx_hbm, out_hbm, idx_v, *bufs_and_sems):
    rows = bufs_and_sems[:_NBUF]
    gsem = bufs_and_sems[_NBUF:2 * _NBUF]
    wsem = bufs_and_sems[2 * _NBUF:3 * _NBUF]

    wid = lax.axis_index("s") * _NC + lax.axis_index("c")
    nchunk = idx_hbm.shape[1]
    pltpu.sync_copy(idx_hbm.at[wid], idx_v)

    def gather_start(c, b):
        pltpu.async_copy(table_hbm.at[idx_v.at[c]], rows[b], gsem[b])

    def gather_wait(c, b):
        pltpu.make_async_copy(table_hbm.at[idx_v.at[c]], rows[b], gsem[b]).wait()

    def wb_start(c, b):
        base = (wid * nchunk + c) * _CHUNK
        pltpu.async_copy(rows[b], out_hbm.at[pl.ds(base, _CHUNK)], wsem[b])

    def wb_wait(c, b):
        base = (wid * nchunk + c) * _CHUNK
        pltpu.make_async_copy(
            rows[b], out_hbm.at[pl.ds(base, _CHUNK)], wsem[b]).wait()

    for b in range(_NBUF):
        gather_start(b, b)

    def outer(c0, carry):
        for b in range(_NBUF):
            c = c0 + b
            gather_wait(c, b)

            def scale_body(r, carry2):
                for j in range(_D // 16):
                    s = pl.ds(j * 16, 16)
                    rows[b][r, s] = rows[b][r, s] * _SCALE
                return carry2

            lax.fori_loop(0, _CHUNK, scale_body, 0, unroll=8)
            wb_start(c, b)
            bp = (b - 1) % _NBUF
            cp = c - 1 + _NBUF

            @pl.when(jnp.logical_and(cp >= _NBUF, cp < nchunk))
            def _():
                wb_wait(cp - _NBUF, bp)
                gather_start(cp, bp)

        return carry

    lax.fori_loop(0, nchunk // _NBUF, lambda i, cr: outer(i * _NBUF, cr), 0)
    for b in range(_NBUF):
        wb_wait(nchunk - _NBUF + b, b)


def kernel(x, emb_weight):
    bt, seq = x.shape
    b = bt * seq
    assert b % (_NW * _CHUNK) == 0
    nchunk = b // (_NW * _CHUNK)
    idx = x.reshape(_NW, nchunk, _CHUNK).astype(jnp.int32)

    mesh = plsc.VectorSubcoreMesh(core_axis_name="c", subcore_axis_name="s")
    f = pl.kernel(
        _emb_body,
        mesh=mesh,
        out_type=jax.ShapeDtypeStruct((b, _D), jnp.float32),
        scratch_types=(
            [pltpu.VMEM((nchunk, _CHUNK), jnp.int32)]
            + [pltpu.VMEM((_CHUNK, _D), jnp.float32) for _ in range(_NBUF)]
            + [pltpu.SemaphoreType.DMA for _ in range(2 * _NBUF)]
        ),
        compiler_params=pltpu.CompilerParams(use_tc_tiling_on_sc=False),
    )
    out = f(emb_weight, idx)
    return out.reshape(bt, seq, _D)
